# Initial kernel scaffold; baseline (speedup 1.0000x reference)
import jax
import jax.numpy as jnp
from jax.experimental import pallas as pl


def kernel(channel_seq, table):
    raise NotImplementedError("write your pallas kernel here")


# speedup vs baseline: 1.3273x; 1.3273x over previous
import jax
import jax.numpy as jnp
from jax import lax
from jax.experimental import pallas as pl
from jax.experimental.pallas import tpu as pltpu
from jax.experimental.pallas import tpu_sc as plsc

DIM = 768
NUM_CORES = 2
NUM_SUBCORES = 16
NUM_WORKERS = NUM_CORES * NUM_SUBCORES
CHUNK = 64


def kernel(channel_seq, table):
    seq_shape = channel_seq.shape
    num_indices = channel_seq.size
    flat_idx = channel_seq.reshape(num_indices).astype(jnp.int32)

    b_per_w = num_indices // NUM_WORKERS
    nchunks = b_per_w // CHUNK
    mesh = plsc.VectorSubcoreMesh(core_axis_name="c", subcore_axis_name="s")

    @jax.jit
    def gather(table, idx):
        @pl.kernel(
            out_type=jax.ShapeDtypeStruct((num_indices, DIM), table.dtype),
            mesh=mesh,
            scratch_types=[
                pltpu.VMEM((b_per_w,), jnp.int32),
                pltpu.VMEM((CHUNK, DIM), jnp.float32),
                pltpu.VMEM((CHUNK, DIM), jnp.float32),
                pltpu.SemaphoreType.DMA,
                pltpu.SemaphoreType.DMA,
            ],
        )
        def sc_gather(table_hbm, idx_hbm, out_hbm, idx_v, rows0, rows1, sem0, sem1):
            wid = lax.axis_index("s") * NUM_CORES + lax.axis_index("c")
            base = wid * b_per_w
            pltpu.sync_copy(idx_hbm.at[pl.ds(base, b_per_w)], idx_v)

            @pl.loop(0, nchunks, step=2)
            def _(g):
                c0 = pltpu.async_copy(
                    table_hbm.at[idx_v.at[pl.ds(g * CHUNK, CHUNK)]], rows0, sem0)
                c1 = pltpu.async_copy(
                    table_hbm.at[idx_v.at[pl.ds((g + 1) * CHUNK, CHUNK)]], rows1, sem1)
                c0.wait()
                pltpu.sync_copy(rows0, out_hbm.at[pl.ds(base + g * CHUNK, CHUNK)])
                c1.wait()
                pltpu.sync_copy(rows1, out_hbm.at[pl.ds(base + (g + 1) * CHUNK, CHUNK)])

        return sc_gather(table, idx)

    out = gather(table, flat_idx)
    return out.reshape(*seq_shape, DIM)


# speedup vs baseline: 1.3339x; 1.0049x over previous
import jax
import jax.numpy as jnp
from jax import lax
from jax.experimental import pallas as pl
from jax.experimental.pallas import tpu as pltpu
from jax.experimental.pallas import tpu_sc as plsc

DIM = 768
NUM_CORES = 2
NUM_SUBCORES = 16
NUM_WORKERS = NUM_CORES * NUM_SUBCORES
CHUNK = 32
NBUF = 4


def kernel(channel_seq, table):
    seq_shape = channel_seq.shape
    num_indices = channel_seq.size
    flat_idx = channel_seq.reshape(num_indices).astype(jnp.int32)

    b_per_w = num_indices // NUM_WORKERS
    nchunks = b_per_w // CHUNK
    mesh = plsc.VectorSubcoreMesh(core_axis_name="c", subcore_axis_name="s")

    @jax.jit
    def gather(table, idx):
        @pl.kernel(
            out_type=jax.ShapeDtypeStruct((num_indices, DIM), table.dtype),
            mesh=mesh,
            scratch_types=(
                [pltpu.VMEM((b_per_w,), jnp.int32)]
                + [pltpu.VMEM((CHUNK, DIM), jnp.float32) for _ in range(NBUF)]
                + [pltpu.SemaphoreType.DMA for _ in range(2 * NBUF)]
            ),
        )
        def sc_gather(table_hbm, idx_hbm, out_hbm, idx_v, *bufs_and_sems):
            rows = bufs_and_sems[:NBUF]
            gsem = bufs_and_sems[NBUF:2 * NBUF]
            ssem = bufs_and_sems[2 * NBUF:]
            wid = lax.axis_index("s") * NUM_CORES + lax.axis_index("c")
            base = wid * b_per_w
            pltpu.sync_copy(idx_hbm.at[pl.ds(base, b_per_w)], idx_v)

            def start_gather(g, b):
                pltpu.async_copy(
                    table_hbm.at[idx_v.at[pl.ds(g * CHUNK, CHUNK)]], rows[b], gsem[b])

            def start_store(g, b):
                pltpu.async_copy(
                    rows[b], out_hbm.at[pl.ds(base + g * CHUNK, CHUNK)], ssem[b])

            def wait_gather(b):
                pltpu.make_async_copy(table_hbm.at[idx_v.at[pl.ds(0, CHUNK)]],
                                      rows[b], gsem[b]).wait()

            def wait_store(b):
                pltpu.make_async_copy(rows[b], out_hbm.at[pl.ds(base, CHUNK)],
                                      ssem[b]).wait()

            K = NBUF // 2

            for b in range(K):
                start_gather(b, b)

            @pl.loop(0, nchunks, step=NBUF)
            def _(g0):
                for b in range(NBUF):
                    g = g0 + b
                    bk = (b + K) % NBUF

                    @pl.when(jnp.logical_and(g + K < nchunks, g + K >= NBUF))
                    def _():
                        wait_store(bk)

                    @pl.when(g + K < nchunks)
                    def _():
                        start_gather(g + K, bk)

                    wait_gather(b)
                    start_store(g, b)

            for b in range(NBUF):
                wait_store(b)

        return sc_gather(table, idx)

    out = gather(table, flat_idx)
    return out.reshape(*seq_shape, DIM)


# speedup vs baseline: 2.1573x; 1.6173x over previous
import jax
import jax.numpy as jnp
from jax import lax
from jax.experimental import pallas as pl
from jax.experimental.pallas import tpu as pltpu
from jax.experimental.pallas import tpu_sc as plsc

DIM = 768
NUM_CORES = 2
NUM_SUBCORES = 16
NUM_WORKERS = NUM_CORES * NUM_SUBCORES
NBUF = 2


def kernel(channel_seq, table):
    nseq, seqlen = channel_seq.shape
    idx2d = channel_seq.astype(jnp.int32)

    seq_per_w = nseq // NUM_WORKERS
    mesh = plsc.VectorSubcoreMesh(core_axis_name="c", subcore_axis_name="s")

    @jax.jit
    def gather(table, idx):
        @pl.kernel(
            out_type=jax.ShapeDtypeStruct((nseq, seqlen, DIM), table.dtype),
            mesh=mesh,
            scratch_types=(
                [pltpu.VMEM((seq_per_w, seqlen), jnp.int32)]
                + [pltpu.VMEM((seqlen, DIM), jnp.float32) for _ in range(NBUF)]
                + [pltpu.SemaphoreType.DMA for _ in range(2 * NBUF)]
            ),
        )
        def sc_gather(table_hbm, idx_hbm, out_hbm, idx_v, *bufs_and_sems):
            rows = bufs_and_sems[:NBUF]
            gsem = bufs_and_sems[NBUF:2 * NBUF]
            ssem = bufs_and_sems[2 * NBUF:]
            wid = lax.axis_index("s") * NUM_CORES + lax.axis_index("c")
            seq_base = wid * seq_per_w
            pltpu.sync_copy(idx_hbm.at[pl.ds(seq_base, seq_per_w)], idx_v)

            def start_gather(s, b):
                pltpu.async_copy(table_hbm.at[idx_v.at[s]], rows[b], gsem[b])

            def start_store(s, b):
                pltpu.async_copy(rows[b], out_hbm.at[seq_base + s], ssem[b])

            def wait_gather(b):
                pltpu.make_async_copy(table_hbm.at[idx_v.at[0]],
                                      rows[b], gsem[b]).wait()

            def wait_store(b):
                pltpu.make_async_copy(rows[b], out_hbm.at[seq_base],
                                      ssem[b]).wait()

            K = 1

            for b in range(K):
                start_gather(b, b)

            @pl.loop(0, seq_per_w, step=NBUF)
            def _(s0):
                for b in range(NBUF):
                    s = s0 + b
                    bk = (b + K) % NBUF

                    @pl.when(jnp.logical_and(s + K < seq_per_w, s + K >= NBUF))
                    def _():
                        wait_store(bk)

                    @pl.when(s + K < seq_per_w)
                    def _():
                        start_gather(s + K, bk)

                    wait_gather(b)
                    start_store(s, b)

            for b in range(NBUF):
                wait_store(b)

        return sc_gather(table, idx)

    return gather(table, idx2d)


# speedup vs baseline: 4.2652x; 1.9771x over previous
import jax
import jax.numpy as jnp
from jax import lax
from jax.experimental import pallas as pl
from jax.experimental.pallas import tpu as pltpu
from jax.experimental.pallas import tpu_sc as plsc

DIM = 768
NUM_CORES = 2
NUM_SUBCORES = 16
NUM_WORKERS = NUM_CORES * NUM_SUBCORES
NBUF = 2
CHUNK = 80


def kernel(channel_seq, table):
    nseq, seqlen = channel_seq.shape
    num_indices = channel_seq.size
    flat_idx = channel_seq.T.reshape(num_indices).astype(jnp.int32)
    b_per_w = num_indices // NUM_WORKERS
    nchunks = b_per_w // CHUNK
    mesh = plsc.VectorSubcoreMesh(core_axis_name="c", subcore_axis_name="s")

    @jax.jit
    def gather(table, idx):
        @pl.kernel(
            out_type=jax.ShapeDtypeStruct((num_indices, DIM), table.dtype),
            mesh=mesh,
            scratch_types=(
                [pltpu.VMEM((b_per_w,), jnp.int32)]
                + [pltpu.VMEM((CHUNK, DIM), jnp.float32) for _ in range(NBUF)]
                + [pltpu.SemaphoreType.DMA for _ in range(2 * NBUF)]
            ),
        )
        def sc_gather(table_hbm, idx_hbm, out_hbm, idx_v, *bufs_and_sems):
            rows = bufs_and_sems[:NBUF]
            gsem = bufs_and_sems[NBUF:2 * NBUF]
            ssem = bufs_and_sems[2 * NBUF:]
            wid = lax.axis_index("s") * NUM_CORES + lax.axis_index("c")
            base = wid * b_per_w
            pltpu.sync_copy(idx_hbm.at[pl.ds(base, b_per_w)], idx_v)

            def start_gather(g, b):
                pltpu.async_copy(
                    table_hbm.at[idx_v.at[pl.ds(g * CHUNK, CHUNK)]], rows[b], gsem[b])

            def start_store(g, b):
                pltpu.async_copy(
                    rows[b], out_hbm.at[pl.ds(base + g * CHUNK, CHUNK)], ssem[b])

            def wait_gather(b):
                pltpu.make_async_copy(table_hbm.at[idx_v.at[pl.ds(0, CHUNK)]],
                                      rows[b], gsem[b]).wait()

            def wait_store(b):
                pltpu.make_async_copy(rows[b], out_hbm.at[pl.ds(base, CHUNK)],
                                      ssem[b]).wait()

            K = 1

            for b in range(K):
                start_gather(b, b)

            @pl.loop(0, nchunks, step=NBUF)
            def _(g0):
                for b in range(NBUF):
                    g = g0 + b
                    bk = (b + K) % NBUF

                    @pl.when(jnp.logical_and(g + K < nchunks, g + K >= NBUF))
                    def _():
                        wait_store(bk)

                    @pl.when(g + K < nchunks)
                    def _():
                        start_gather(g + K, bk)

                    wait_gather(b)
                    start_store(g, b)

            for b in range(NBUF):
                wait_store(b)

        return sc_gather(table, idx)

    out = gather(table, flat_idx)
    return out.reshape(seqlen, nseq, DIM).transpose(1, 0, 2)


# speedup vs baseline: 4.2738x; 1.0020x over previous
import jax
import jax.numpy as jnp
from jax import lax
from jax.experimental import pallas as pl
from jax.experimental.pallas import tpu as pltpu
from jax.experimental.pallas import tpu_sc as plsc

DIM = 768
NUM_CORES = 2
NUM_SUBCORES = 16
NUM_WORKERS = NUM_CORES * NUM_SUBCORES
NBUF = 4
CHUNK = 32


def kernel(channel_seq, table):
    nseq, seqlen = channel_seq.shape
    num_indices = channel_seq.size
    flat_idx = channel_seq.T.reshape(num_indices).astype(jnp.int32)
    b_per_w = num_indices // NUM_WORKERS
    nchunks = b_per_w // CHUNK
    mesh = plsc.VectorSubcoreMesh(core_axis_name="c", subcore_axis_name="s")

    @jax.jit
    def gather(table, idx):
        @pl.kernel(
            out_type=jax.ShapeDtypeStruct((num_indices, DIM), table.dtype),
            mesh=mesh,
            scratch_types=(
                [pltpu.VMEM((b_per_w,), jnp.int32)]
                + [pltpu.VMEM((CHUNK, DIM), jnp.float32) for _ in range(NBUF)]
                + [pltpu.SemaphoreType.DMA for _ in range(2 * NBUF)]
            ),
        )
        def sc_gather(table_hbm, idx_hbm, out_hbm, idx_v, *bufs_and_sems):
            rows = bufs_and_sems[:NBUF]
            gsem = bufs_and_sems[NBUF:2 * NBUF]
            ssem = bufs_and_sems[2 * NBUF:]
            wid = lax.axis_index("s") * NUM_CORES + lax.axis_index("c")
            base = wid * b_per_w
            pltpu.sync_copy(idx_hbm.at[pl.ds(base, b_per_w)], idx_v)

            def start_gather(g, b):
                pltpu.async_copy(
                    table_hbm.at[idx_v.at[pl.ds(g * CHUNK, CHUNK)]], rows[b], gsem[b])

            def start_store(g, b):
                pltpu.async_copy(
                    rows[b], out_hbm.at[pl.ds(base + g * CHUNK, CHUNK)], ssem[b])

            def wait_gather(b):
                pltpu.make_async_copy(table_hbm.at[idx_v.at[pl.ds(0, CHUNK)]],
                                      rows[b], gsem[b]).wait()

            def wait_store(b):
                pltpu.make_async_copy(rows[b], out_hbm.at[pl.ds(base, CHUNK)],
                                      ssem[b]).wait()

            K = 1

            for b in range(K):
                start_gather(b, b)

            @pl.loop(0, nchunks, step=NBUF)
            def _(g0):
                for b in range(NBUF):
                    g = g0 + b
                    bk = (b + K) % NBUF

                    @pl.when(jnp.logical_and(g + K < nchunks, g + K >= NBUF))
                    def _():
                        wait_store(bk)

                    @pl.when(g + K < nchunks)
                    def _():
                        start_gather(g + K, bk)

                    wait_gather(b)
                    start_store(g, b)

            for b in range(NBUF):
                wait_store(b)

        return sc_gather(table, idx)

    out = gather(table, flat_idx)
    return out.reshape(seqlen, nseq, DIM).transpose(1, 0, 2)


# speedup vs baseline: 4.2782x; 1.0010x over previous
import jax
import jax.numpy as jnp
from jax import lax
from jax.experimental import pallas as pl
from jax.experimental.pallas import tpu as pltpu
from jax.experimental.pallas import tpu_sc as plsc

DIM = 768
NUM_CORES = 2
NUM_SUBCORES = 16
NUM_WORKERS = NUM_CORES * NUM_SUBCORES
NBUF = 4
CHUNK = 32


def kernel(channel_seq, table):
    nseq, seqlen = channel_seq.shape
    num_indices = channel_seq.size
    flat_idx = channel_seq.T.reshape(num_indices).astype(jnp.int32)
    b_per_w = num_indices // NUM_WORKERS
    nchunks = b_per_w // CHUNK
    mesh = plsc.VectorSubcoreMesh(core_axis_name="c", subcore_axis_name="s")

    @jax.jit
    def gather(table, idx):
        @pl.kernel(
            out_type=jax.ShapeDtypeStruct((num_indices, DIM), table.dtype),
            mesh=mesh,
            scratch_types=(
                [pltpu.VMEM((b_per_w,), jnp.int32)]
                + [pltpu.VMEM((CHUNK, DIM), jnp.float32) for _ in range(NBUF)]
                + [pltpu.SemaphoreType.DMA for _ in range(2 * NBUF)]
            ),
        )
        def sc_gather(table_hbm, idx_hbm, out_hbm, idx_v, *bufs_and_sems):
            rows = bufs_and_sems[:NBUF]
            gsem = bufs_and_sems[NBUF:2 * NBUF]
            ssem = bufs_and_sems[2 * NBUF:]
            wid = lax.axis_index("s") * NUM_CORES + lax.axis_index("c")
            base = wid * b_per_w
            pltpu.sync_copy(idx_hbm.at[pl.ds(base, b_per_w)], idx_v)

            def start_gather(g, b):
                pltpu.async_copy(
                    table_hbm.at[idx_v.at[pl.ds(g * CHUNK, CHUNK)]], rows[b], gsem[b])

            def start_store(g, b):
                pltpu.async_copy(
                    rows[b], out_hbm.at[pl.ds(base + g * CHUNK, CHUNK)], ssem[b])

            def wait_gather(b):
                pltpu.make_async_copy(table_hbm.at[idx_v.at[pl.ds(0, CHUNK)]],
                                      rows[b], gsem[b]).wait()

            def wait_store(b):
                pltpu.make_async_copy(rows[b], out_hbm.at[pl.ds(base, CHUNK)],
                                      ssem[b]).wait()

            K = 3

            for b in range(K):
                start_gather(b, b)

            @pl.loop(0, nchunks, step=NBUF)
            def _(g0):
                for b in range(NBUF):
                    g = g0 + b
                    bk = (b + K) % NBUF

                    @pl.when(jnp.logical_and(g + K < nchunks, g + K >= NBUF))
                    def _():
                        wait_store(bk)

                    @pl.when(g + K < nchunks)
                    def _():
                        start_gather(g + K, bk)

                    wait_gather(b)
                    start_store(g, b)

            for b in range(NBUF):
                wait_store(b)

        return sc_gather(table, idx)

    out = gather(table, flat_idx)
    return out.reshape(seqlen, nseq, DIM).transpose(1, 0, 2)
